# Initial kernel scaffold; baseline (speedup 1.0000x reference)
import jax
import jax.numpy as jnp
from jax.experimental import pallas as pl


def kernel(buy_src, buy_dst, cart_src, cart_dst, pv_src, pv_dst, e_type, user_emb, item_emb, edges_emb, W1, W2, W3, W4, EW1, EW2, EW3, EW4):
    raise NotImplementedError("write your pallas kernel here")


# speedup vs baseline: 1.1622x; 1.1622x over previous
import functools

import jax
import jax.numpy as jnp
from jax.experimental import pallas as pl

_NEG_SLOPE = 0.01
_W_BUY = 1.0 / 6.0
_W_CART = 5.0 / 6.0
_ROW_BLK = 1024


def _combine_body(ab_ref, ac_ref, wb_ref, wc_ref, o_ref):
    xb = ab_ref[...] @ wb_ref[...]
    xc = ac_ref[...] @ wc_ref[...]
    yb = jnp.where(xb >= 0, xb, _NEG_SLOPE * xb)
    yc = jnp.where(xc >= 0, xc, _NEG_SLOPE * xc)
    o_ref[...] = _W_BUY * yb + _W_CART * yc


def _fused_combine(ab, ac, wb, wc):
    n, d = ab.shape
    grid = n // _ROW_BLK
    return pl.pallas_call(
        _combine_body,
        grid=(grid,),
        in_specs=[
            pl.BlockSpec((_ROW_BLK, d), lambda i: (i, 0)),
            pl.BlockSpec((_ROW_BLK, d), lambda i: (i, 0)),
            pl.BlockSpec((d, d), lambda i: (0, 0)),
            pl.BlockSpec((d, d), lambda i: (0, 0)),
        ],
        out_specs=pl.BlockSpec((_ROW_BLK, d), lambda i: (i, 0)),
        out_shape=jax.ShapeDtypeStruct((n, d), jnp.float32),
    )(ab, ac, wb, wc)


def kernel(buy_src, buy_dst, cart_src, cart_dst, pv_src, pv_dst, e_type,
           user_emb, item_emb, edges_emb, W1, W2, W3, W4, EW1, EW2, EW3, EW4):
    nu, d = user_emb.shape
    ni = item_emb.shape[0]
    sf = user_emb * 0.01
    df = item_emb * 0.01
    ef = edges_emb[e_type] * 0.01
    vb, vc = ef[0], ef[2]

    ones = jnp.ones(buy_src.shape[0], jnp.float32)
    duB = jnp.maximum(jax.ops.segment_sum(ones, buy_src, num_segments=nu), 1.0)
    diB = jnp.maximum(jax.ops.segment_sum(ones, buy_dst, num_segments=ni), 1.0)
    duC = jnp.maximum(jax.ops.segment_sum(ones, cart_src, num_segments=nu), 1.0)
    diC = jnp.maximum(jax.ops.segment_sum(ones, cart_dst, num_segments=ni), 1.0)
    wB = jax.lax.rsqrt(duB)[buy_src] * jax.lax.rsqrt(diB)[buy_dst]
    wC = jax.lax.rsqrt(duC)[cart_src] * jax.lax.rsqrt(diC)[cart_dst]

    n_tot = nu + ni
    n_pad = ((n_tot + _ROW_BLK - 1) // _ROW_BLK) * _ROW_BLK

    for W, EW in ((W1, EW1), (W2, EW2), (W3, EW3), (W4, EW4)):
        wb_f = vb[:, None] * W
        wc_f = vc[:, None] * W
        aggU_b = jax.ops.segment_sum(wB[:, None] * df[buy_dst], buy_src,
                                     num_segments=nu)
        aggI_b = jax.ops.segment_sum(wB[:, None] * sf[buy_src], buy_dst,
                                     num_segments=ni)
        aggU_c = jax.ops.segment_sum(wC[:, None] * df[cart_dst], cart_src,
                                     num_segments=nu)
        aggI_c = jax.ops.segment_sum(wC[:, None] * sf[cart_src], cart_dst,
                                     num_segments=ni)
        ab = jnp.concatenate([aggU_b, aggI_b,
                              jnp.zeros((n_pad - n_tot, d), jnp.float32)])
        ac = jnp.concatenate([aggU_c, aggI_c,
                              jnp.zeros((n_pad - n_tot, d), jnp.float32)])
        out = _fused_combine(ab, ac, wb_f, wc_f)
        sf, df = out[:nu], out[nu:n_tot]
        vb, vc = vb @ EW, vc @ EW
    return sf, df


# speedup vs baseline: 1.1806x; 1.0159x over previous
import functools

import jax
import jax.numpy as jnp
from jax import lax
from jax.experimental import pallas as pl
from jax.experimental.pallas import tpu as pltpu
from jax.experimental.pallas import tpu_sc as plsc

_NEG_SLOPE = 0.01
_W_BUY = 1.0 / 6.0
_W_CART = 5.0 / 6.0
_ROW_BLK = 1024

_E_CHUNK = 128
_N_SUBCORES = 16
_N_CORES = 2
_SLAB = 12544
_STRIPE = _SLAB // _N_SUBCORES
_ZROWS = 56


def _sc_agg(table, gidx, sidx, zeros_h, n_passes):
    d = table.shape[1]
    e = gidx.shape[0]
    n_chunks = e // _E_CHUNK
    ub = -(-n_chunks // _N_SUBCORES)
    out_rows = n_passes * _N_CORES * _SLAB
    mesh = plsc.VectorSubcoreMesh(core_axis_name="c", subcore_axis_name="s")

    def body(table_h, gidx_h, sidx_h, zv_h, out_h,
             srcb, dstb, locb, rows, zbuf, slab, sem):
        c = lax.axis_index("c")
        s = lax.axis_index("s")
        pltpu.sync_copy(zv_h, zbuf)
        for p in range(n_passes):
            base = (_N_CORES * p + c) * _SLAB
            for z in range(_STRIPE // _ZROWS):
                pltpu.sync_copy(zbuf,
                                slab.at[pl.ds(s * _STRIPE + z * _ZROWS,
                                              _ZROWS)])
            plsc.subcore_barrier()

            def chunk_body(i, carry):
                g = i * _N_SUBCORES + s

                @pl.when(g < n_chunks)
                def _():
                    off = g * _E_CHUNK
                    pltpu.sync_copy(gidx_h.at[pl.ds(off, _E_CHUNK)], srcb)
                    pltpu.sync_copy(sidx_h.at[pl.ds(off, _E_CHUNK)], dstb)
                    for v in range(_E_CHUNK // 16):
                        dv = dstb[pl.ds(v * 16, 16)]
                        loc = dv - base
                        ok = (loc >= 0) & (loc < _SLAB)
                        locb[pl.ds(v * 16, 16)] = jnp.where(ok, loc, _SLAB)
                    pltpu.async_copy(table_h.at[srcb], rows, sem).wait()
                    pltpu.sync_copy(rows, slab.at[locb], add=True)
                return carry

            lax.fori_loop(0, ub, chunk_body, 0)
            plsc.subcore_barrier()
            pltpu.sync_copy(slab.at[pl.ds(s * _STRIPE, _STRIPE)],
                            out_h.at[pl.ds(base + s * _STRIPE, _STRIPE)])
            plsc.subcore_barrier()

    return pl.kernel(
        body,
        out_type=jax.ShapeDtypeStruct((out_rows, d), jnp.float32),
        mesh=mesh,
        scratch_types=[
            pltpu.VMEM((_E_CHUNK,), jnp.int32),
            pltpu.VMEM((_E_CHUNK,), jnp.int32),
            pltpu.VMEM((_E_CHUNK,), jnp.int32),
            pltpu.VMEM((_E_CHUNK, d), jnp.float32),
            pltpu.VMEM((_ZROWS, d), jnp.float32),
            pltpu.VMEM_SHARED((_SLAB + 8, d), jnp.float32),
            pltpu.SemaphoreType.DMA,
        ],
    )(table, gidx, sidx, zeros_h)


def _lrelu(x):
    return jnp.where(x >= 0, x, _NEG_SLOPE * x)


def _combine2_body(ab_ref, ac_ref, rb_ref, rc_ref, wb_ref, wc_ref,
                   ob_ref, oc_ref):
    rb = rb_ref[...]
    rc = rc_ref[...]
    xb = (ab_ref[...] * rb) @ wb_ref[...]
    xc = (ac_ref[...] * rc) @ wc_ref[...]
    t = _W_BUY * _lrelu(xb) + _W_CART * _lrelu(xc)
    ob_ref[...] = t * rb
    oc_ref[...] = t * rc


def _combine1_body(ab_ref, ac_ref, rb_ref, rc_ref, wb_ref, wc_ref, o_ref):
    xb = (ab_ref[...] * rb_ref[...]) @ wb_ref[...]
    xc = (ac_ref[...] * rc_ref[...]) @ wc_ref[...]
    o_ref[...] = _W_BUY * _lrelu(xb) + _W_CART * _lrelu(xc)


def _combine(ab, ac, rb, rc, wb, wc, two_outputs):
    n, d = ab.shape
    grid = n // _ROW_BLK
    row_spec = pl.BlockSpec((_ROW_BLK, d), lambda i: (i, 0))
    col_spec = pl.BlockSpec((_ROW_BLK, 1), lambda i: (i, 0))
    w_spec = pl.BlockSpec((d, d), lambda i: (0, 0))
    out_sd = jax.ShapeDtypeStruct((n, d), jnp.float32)
    if two_outputs:
        return pl.pallas_call(
            _combine2_body,
            grid=(grid,),
            in_specs=[row_spec, row_spec, col_spec, col_spec, w_spec, w_spec],
            out_specs=[row_spec, row_spec],
            out_shape=[out_sd, out_sd],
        )(ab, ac, rb, rc, wb, wc)
    return pl.pallas_call(
        _combine1_body,
        grid=(grid,),
        in_specs=[row_spec, row_spec, col_spec, col_spec, w_spec, w_spec],
        out_specs=row_spec,
        out_shape=out_sd,
    )(ab, ac, rb, rc, wb, wc)


def _scale2_body(e_ref, rb_ref, rc_ref, ob_ref, oc_ref):
    x = e_ref[...] * 0.01
    ob_ref[...] = x * rb_ref[...]
    oc_ref[...] = x * rc_ref[...]


def _scale2(emb, rb, rc):
    n, d = emb.shape
    grid = n // _ROW_BLK
    row_spec = pl.BlockSpec((_ROW_BLK, d), lambda i: (i, 0))
    col_spec = pl.BlockSpec((_ROW_BLK, 1), lambda i: (i, 0))
    out_sd = jax.ShapeDtypeStruct((n, d), jnp.float32)
    return pl.pallas_call(
        _scale2_body,
        grid=(grid,),
        in_specs=[row_spec, col_spec, col_spec],
        out_specs=[row_spec, row_spec],
        out_shape=[out_sd, out_sd],
    )(emb, rb, rc)


def kernel(buy_src, buy_dst, cart_src, cart_dst, pv_src, pv_dst, e_type,
           user_emb, item_emb, edges_emb, W1, W2, W3, W4, EW1, EW2, EW3, EW4):
    nu, d = user_emb.shape
    ni = item_emb.shape[0]
    ef = edges_emb[e_type] * 0.01
    vb, vc = ef[0], ef[2]

    pu = 4 * _N_CORES * _SLAB
    pi = 2 * _N_CORES * _SLAB
    assert pu >= nu and pi >= ni and pu % _ROW_BLK == 0 and pi % _ROW_BLK == 0

    ones = jnp.ones(buy_src.shape[0], jnp.float32)
    duB = jnp.maximum(jax.ops.segment_sum(ones, buy_src, num_segments=nu), 1.0)
    diB = jnp.maximum(jax.ops.segment_sum(ones, buy_dst, num_segments=ni), 1.0)
    duC = jnp.maximum(jax.ops.segment_sum(ones, cart_src, num_segments=nu), 1.0)
    diC = jnp.maximum(jax.ops.segment_sum(ones, cart_dst, num_segments=ni), 1.0)
    rbu = jnp.pad(lax.rsqrt(duB), (0, pu - nu), constant_values=1.0)[:, None]
    rcu = jnp.pad(lax.rsqrt(duC), (0, pu - nu), constant_values=1.0)[:, None]
    rbi = jnp.pad(lax.rsqrt(diB), (0, pi - ni), constant_values=1.0)[:, None]
    rci = jnp.pad(lax.rsqrt(diC), (0, pi - ni), constant_values=1.0)[:, None]

    eu = jnp.pad(user_emb, ((0, pu - nu), (0, 0)))
    ei = jnp.pad(item_emb, ((0, pi - ni), (0, 0)))
    sfB, sfC = _scale2(eu, rbu, rcu)
    dfB, dfC = _scale2(ei, rbi, rci)

    zeros_h = jnp.zeros((_ZROWS, d), jnp.float32)

    out_u = out_i = None
    for li, (W, EW) in enumerate(((W1, EW1), (W2, EW2), (W3, EW3), (W4, EW4))):
        wb_f = vb[:, None] * W
        wc_f = vc[:, None] * W
        aggU_b = _sc_agg(dfB, buy_dst, buy_src, zeros_h, 4)
        aggI_b = _sc_agg(sfB, buy_src, buy_dst, zeros_h, 2)
        aggU_c = _sc_agg(dfC, cart_dst, cart_src, zeros_h, 4)
        aggI_c = _sc_agg(sfC, cart_src, cart_dst, zeros_h, 2)
        if li < 3:
            sfB, sfC = _combine(aggU_b, aggU_c, rbu, rcu, wb_f, wc_f, True)
            dfB, dfC = _combine(aggI_b, aggI_c, rbi, rci, wb_f, wc_f, True)
            vb, vc = vb @ EW, vc @ EW
        else:
            out_u = _combine(aggU_b, aggU_c, rbu, rcu, wb_f, wc_f, False)
            out_i = _combine(aggI_b, aggI_c, rbi, rci, wb_f, wc_f, False)
    return out_u[:nu], out_i[:ni]


# speedup vs baseline: 1.4665x; 1.2421x over previous
import functools

import jax
import jax.numpy as jnp
from jax import lax
from jax.experimental import pallas as pl
from jax.experimental.pallas import tpu as pltpu
from jax.experimental.pallas import tpu_sc as plsc

_NEG_SLOPE = 0.01
_W_BUY = 1.0 / 6.0
_W_CART = 5.0 / 6.0
_ROW_BLK = 1024

_E_CHUNK = 80
_N_SUBCORES = 16
_N_CORES = 2
_SLAB = 12544
_STRIPE = _SLAB // _N_SUBCORES


def _sc_agg(table, gidx, sidx, zeros_h, n_passes):
    d = table.shape[1]
    e = gidx.shape[0]
    n_chunks = e // _E_CHUNK
    jmax = -(-n_chunks // _N_SUBCORES)
    out_rows = n_passes * _N_CORES * _SLAB
    mesh = plsc.VectorSubcoreMesh(core_axis_name="c", subcore_axis_name="s")

    def body(table_h, gidx_h, sidx_h, zv_h, out_h,
             src0, src1, dst0, dst1, loc0, loc1, rows0, rows1,
             slab, sem0, sem1):
        c = lax.axis_index("c")
        s = lax.axis_index("s")
        srcs, dsts, locs = (src0, src1), (dst0, dst1), (loc0, loc1)
        rows, sems = (rows0, rows1), (sem0, sem1)
        for p in range(n_passes):
            base = (_N_CORES * p + c) * _SLAB
            pltpu.sync_copy(zv_h, slab.at[pl.ds(s * _STRIPE, _STRIPE)])
            plsc.subcore_barrier()

            def issue(b, j):
                @pl.when(j < jmax)
                def _():
                    g = j * _N_SUBCORES + s
                    off = g * _E_CHUNK
                    pltpu.sync_copy(gidx_h.at[pl.ds(off, _E_CHUNK)], srcs[b])
                    pltpu.sync_copy(sidx_h.at[pl.ds(off, _E_CHUNK)], dsts[b])
                    for v in range(_E_CHUNK // 16):
                        dv = dsts[b][pl.ds(v * 16, 16)]
                        loc = dv - base
                        ok = (loc >= 0) & (loc < _SLAB)
                        locs[b][pl.ds(v * 16, 16)] = jnp.where(ok, loc, _SLAB)
                    pltpu.async_copy(table_h.at[srcs[b]], rows[b], sems[b])

            def drain(b, j):
                @pl.when(j < jmax)
                def _():
                    pltpu.make_async_copy(table_h.at[srcs[b]], rows[b],
                                          sems[b]).wait()
                    pltpu.sync_copy(rows[b], slab.at[locs[b]], add=True)

            issue(0, 0)

            def pair(k, carry):
                j0 = k * 2
                issue(1, j0 + 1)
                drain(0, j0)
                issue(0, j0 + 2)
                drain(1, j0 + 1)
                return carry

            lax.fori_loop(0, (jmax + 1) // 2, pair, 0)
            plsc.subcore_barrier()
            pltpu.sync_copy(slab.at[pl.ds(s * _STRIPE, _STRIPE)],
                            out_h.at[pl.ds(base + s * _STRIPE, _STRIPE)])
            plsc.subcore_barrier()

    return pl.kernel(
        body,
        out_type=jax.ShapeDtypeStruct((out_rows, d), jnp.float32),
        mesh=mesh,
        scratch_types=[
            pltpu.VMEM((_E_CHUNK,), jnp.int32),
            pltpu.VMEM((_E_CHUNK,), jnp.int32),
            pltpu.VMEM((_E_CHUNK,), jnp.int32),
            pltpu.VMEM((_E_CHUNK,), jnp.int32),
            pltpu.VMEM((_E_CHUNK,), jnp.int32),
            pltpu.VMEM((_E_CHUNK,), jnp.int32),
            pltpu.VMEM((_E_CHUNK, d), jnp.float32),
            pltpu.VMEM((_E_CHUNK, d), jnp.float32),
            pltpu.VMEM_SHARED((_SLAB + 8, d), jnp.float32),
            pltpu.SemaphoreType.DMA,
            pltpu.SemaphoreType.DMA,
        ],
    )(table, gidx, sidx, zeros_h)


def _lrelu(x):
    return jnp.where(x >= 0, x, _NEG_SLOPE * x)


def _combine2_body(ab_ref, ac_ref, rb_ref, rc_ref, wb_ref, wc_ref,
                   ob_ref, oc_ref):
    rb = rb_ref[...]
    rc = rc_ref[...]
    xb = (ab_ref[...] * rb) @ wb_ref[...]
    xc = (ac_ref[...] * rc) @ wc_ref[...]
    t = _W_BUY * _lrelu(xb) + _W_CART * _lrelu(xc)
    ob_ref[...] = t * rb
    oc_ref[...] = t * rc


def _combine1_body(ab_ref, ac_ref, rb_ref, rc_ref, wb_ref, wc_ref, o_ref):
    xb = (ab_ref[...] * rb_ref[...]) @ wb_ref[...]
    xc = (ac_ref[...] * rc_ref[...]) @ wc_ref[...]
    o_ref[...] = _W_BUY * _lrelu(xb) + _W_CART * _lrelu(xc)


def _combine(ab, ac, rb, rc, wb, wc, two_outputs):
    n, d = ab.shape
    grid = n // _ROW_BLK
    row_spec = pl.BlockSpec((_ROW_BLK, d), lambda i: (i, 0))
    col_spec = pl.BlockSpec((_ROW_BLK, 1), lambda i: (i, 0))
    w_spec = pl.BlockSpec((d, d), lambda i: (0, 0))
    out_sd = jax.ShapeDtypeStruct((n, d), jnp.float32)
    if two_outputs:
        return pl.pallas_call(
            _combine2_body,
            grid=(grid,),
            in_specs=[row_spec, row_spec, col_spec, col_spec, w_spec, w_spec],
            out_specs=[row_spec, row_spec],
            out_shape=[out_sd, out_sd],
        )(ab, ac, rb, rc, wb, wc)
    return pl.pallas_call(
        _combine1_body,
        grid=(grid,),
        in_specs=[row_spec, row_spec, col_spec, col_spec, w_spec, w_spec],
        out_specs=row_spec,
        out_shape=out_sd,
    )(ab, ac, rb, rc, wb, wc)


def _scale2_body(e_ref, rb_ref, rc_ref, ob_ref, oc_ref):
    x = e_ref[...] * 0.01
    ob_ref[...] = x * rb_ref[...]
    oc_ref[...] = x * rc_ref[...]


def _scale2(emb, rb, rc):
    n, d = emb.shape
    grid = n // _ROW_BLK
    row_spec = pl.BlockSpec((_ROW_BLK, d), lambda i: (i, 0))
    col_spec = pl.BlockSpec((_ROW_BLK, 1), lambda i: (i, 0))
    out_sd = jax.ShapeDtypeStruct((n, d), jnp.float32)
    return pl.pallas_call(
        _scale2_body,
        grid=(grid,),
        in_specs=[row_spec, col_spec, col_spec],
        out_specs=[row_spec, row_spec],
        out_shape=[out_sd, out_sd],
    )(emb, rb, rc)


def kernel(buy_src, buy_dst, cart_src, cart_dst, pv_src, pv_dst, e_type,
           user_emb, item_emb, edges_emb, W1, W2, W3, W4, EW1, EW2, EW3, EW4):
    nu, d = user_emb.shape
    ni = item_emb.shape[0]
    ef = edges_emb[e_type] * 0.01
    vb, vc = ef[0], ef[2]

    pu = 4 * _N_CORES * _SLAB
    pi = 2 * _N_CORES * _SLAB
    assert pu >= nu and pi >= ni and pu % _ROW_BLK == 0 and pi % _ROW_BLK == 0

    ones = jnp.ones(buy_src.shape[0], jnp.float32)
    duB = jnp.maximum(jax.ops.segment_sum(ones, buy_src, num_segments=nu), 1.0)
    diB = jnp.maximum(jax.ops.segment_sum(ones, buy_dst, num_segments=ni), 1.0)
    duC = jnp.maximum(jax.ops.segment_sum(ones, cart_src, num_segments=nu), 1.0)
    diC = jnp.maximum(jax.ops.segment_sum(ones, cart_dst, num_segments=ni), 1.0)
    rbu = jnp.pad(lax.rsqrt(duB), (0, pu - nu), constant_values=1.0)[:, None]
    rcu = jnp.pad(lax.rsqrt(duC), (0, pu - nu), constant_values=1.0)[:, None]
    rbi = jnp.pad(lax.rsqrt(diB), (0, pi - ni), constant_values=1.0)[:, None]
    rci = jnp.pad(lax.rsqrt(diC), (0, pi - ni), constant_values=1.0)[:, None]

    eu = jnp.pad(user_emb, ((0, pu - nu), (0, 0)))
    ei = jnp.pad(item_emb, ((0, pi - ni), (0, 0)))
    sfB, sfC = _scale2(eu, rbu, rcu)
    dfB, dfC = _scale2(ei, rbi, rci)

    zeros_h = jnp.zeros((_STRIPE, d), jnp.float32)

    out_u = out_i = None
    for li, (W, EW) in enumerate(((W1, EW1), (W2, EW2), (W3, EW3), (W4, EW4))):
        wb_f = vb[:, None] * W
        wc_f = vc[:, None] * W
        aggU_b = _sc_agg(dfB, buy_dst, buy_src, zeros_h, 4)
        aggI_b = _sc_agg(sfB, buy_src, buy_dst, zeros_h, 2)
        aggU_c = _sc_agg(dfC, cart_dst, cart_src, zeros_h, 4)
        aggI_c = _sc_agg(sfC, cart_src, cart_dst, zeros_h, 2)
        if li < 3:
            sfB, sfC = _combine(aggU_b, aggU_c, rbu, rcu, wb_f, wc_f, True)
            dfB, dfC = _combine(aggI_b, aggI_c, rbi, rci, wb_f, wc_f, True)
            vb, vc = vb @ EW, vc @ EW
        else:
            out_u = _combine(aggU_b, aggU_c, rbu, rcu, wb_f, wc_f, False)
            out_i = _combine(aggI_b, aggI_c, rbi, rci, wb_f, wc_f, False)
    return out_u[:nu], out_i[:ni]


# speedup vs baseline: 1.5551x; 1.0604x over previous
import functools

import jax
import jax.numpy as jnp
from jax import lax
from jax.experimental import pallas as pl
from jax.experimental.pallas import tpu as pltpu
from jax.experimental.pallas import tpu_sc as plsc

_NEG_SLOPE = 0.01
_W_BUY = 1.0 / 6.0
_W_CART = 5.0 / 6.0
_ROW_BLK = 1024

_E_CHUNK = 80
_N_SUBCORES = 16
_N_CORES = 2
_SLAB = 12544
_STRIPE = _SLAB // _N_SUBCORES


def _sc_agg(table, gidx, sidx, zeros_h, n_passes):
    d = table.shape[1]
    e = gidx.shape[0]
    n_chunks = e // _E_CHUNK
    jmax = -(-n_chunks // _N_SUBCORES)
    out_rows = n_passes * _N_CORES * _SLAB
    mesh = plsc.VectorSubcoreMesh(core_axis_name="c", subcore_axis_name="s")

    def body(table_h, gidx_h, sidx_h, zv_h, out_h,
             src0, src1, src2, dst0, dst1, dst2, loc0, loc1, loc2,
             rows0, rows1, slab, isem0, isem1, isem2, gsem0, gsem1):
        c = lax.axis_index("c")
        s = lax.axis_index("s")
        srcs, dsts, locs = (src0, src1, src2), (dst0, dst1, dst2), \
            (loc0, loc1, loc2)
        rows, isems, gsems = (rows0, rows1), (isem0, isem1, isem2), \
            (gsem0, gsem1)
        for p in range(n_passes):
            base = (_N_CORES * p + c) * _SLAB
            pltpu.sync_copy(zv_h, slab.at[pl.ds(s * _STRIPE, _STRIPE)])
            plsc.subcore_barrier()

            def idx_issue(j, b):
                @pl.when(j < jmax)
                def _():
                    g = j * _N_SUBCORES + s
                    off = g * _E_CHUNK
                    pltpu.async_copy(gidx_h.at[pl.ds(off, _E_CHUNK)],
                                     srcs[b], isems[b])
                    pltpu.async_copy(sidx_h.at[pl.ds(off, _E_CHUNK)],
                                     dsts[b], isems[b])

            def gather_issue(j, b, b2):
                @pl.when(j < jmax)
                def _():
                    g = j * _N_SUBCORES + s
                    off = g * _E_CHUNK
                    pltpu.make_async_copy(gidx_h.at[pl.ds(off, _E_CHUNK)],
                                          srcs[b], isems[b]).wait()
                    pltpu.make_async_copy(sidx_h.at[pl.ds(off, _E_CHUNK)],
                                          dsts[b], isems[b]).wait()
                    for v in range(_E_CHUNK // 16):
                        dv = dsts[b][pl.ds(v * 16, 16)]
                        loc = dv - base
                        ok = (loc >= 0) & (loc < _SLAB)
                        locs[b][pl.ds(v * 16, 16)] = jnp.where(ok, loc, _SLAB)
                    pltpu.async_copy(table_h.at[srcs[b]], rows[b2], gsems[b2])

            def drain(j, b, b2):
                @pl.when(j < jmax)
                def _():
                    pltpu.make_async_copy(table_h.at[srcs[b]], rows[b2],
                                          gsems[b2]).wait()
                    pltpu.sync_copy(rows[b2], slab.at[locs[b]], add=True)

            idx_issue(0, 0)
            idx_issue(1, 1)
            gather_issue(0, 0, 0)
            idx_issue(2, 2)

            def six(k, carry):
                j0 = k * 6
                for t in range(6):
                    gather_issue(j0 + t + 1, (t + 1) % 3, (t + 1) % 2)
                    drain(j0 + t, t % 3, t % 2)
                    idx_issue(j0 + t + 3, t % 3)
                return carry

            lax.fori_loop(0, -(-jmax // 6), six, 0)
            plsc.subcore_barrier()
            pltpu.sync_copy(slab.at[pl.ds(s * _STRIPE, _STRIPE)],
                            out_h.at[pl.ds(base + s * _STRIPE, _STRIPE)])
            plsc.subcore_barrier()

    return pl.kernel(
        body,
        out_type=jax.ShapeDtypeStruct((out_rows, d), jnp.float32),
        mesh=mesh,
        scratch_types=(
            [pltpu.VMEM((_E_CHUNK,), jnp.int32)] * 9
            + [pltpu.VMEM((_E_CHUNK, d), jnp.float32)] * 2
            + [pltpu.VMEM_SHARED((_SLAB + 8, d), jnp.float32)]
            + [pltpu.SemaphoreType.DMA] * 5
        ),
    )(table, gidx, sidx, zeros_h)


def _lrelu(x):
    return jnp.where(x >= 0, x, _NEG_SLOPE * x)


def _combine2_body(ab_ref, ac_ref, rb_ref, rc_ref, wb_ref, wc_ref,
                   ob_ref, oc_ref):
    rb = rb_ref[...]
    rc = rc_ref[...]
    xb = (ab_ref[...] * rb) @ wb_ref[...]
    xc = (ac_ref[...] * rc) @ wc_ref[...]
    t = _W_BUY * _lrelu(xb) + _W_CART * _lrelu(xc)
    ob_ref[...] = t * rb
    oc_ref[...] = t * rc


def _combine1_body(ab_ref, ac_ref, rb_ref, rc_ref, wb_ref, wc_ref, o_ref):
    xb = (ab_ref[...] * rb_ref[...]) @ wb_ref[...]
    xc = (ac_ref[...] * rc_ref[...]) @ wc_ref[...]
    o_ref[...] = _W_BUY * _lrelu(xb) + _W_CART * _lrelu(xc)


def _combine(ab, ac, rb, rc, wb, wc, two_outputs):
    n, d = ab.shape
    grid = n // _ROW_BLK
    row_spec = pl.BlockSpec((_ROW_BLK, d), lambda i: (i, 0))
    col_spec = pl.BlockSpec((_ROW_BLK, 1), lambda i: (i, 0))
    w_spec = pl.BlockSpec((d, d), lambda i: (0, 0))
    out_sd = jax.ShapeDtypeStruct((n, d), jnp.float32)
    if two_outputs:
        return pl.pallas_call(
            _combine2_body,
            grid=(grid,),
            in_specs=[row_spec, row_spec, col_spec, col_spec, w_spec, w_spec],
            out_specs=[row_spec, row_spec],
            out_shape=[out_sd, out_sd],
        )(ab, ac, rb, rc, wb, wc)
    return pl.pallas_call(
        _combine1_body,
        grid=(grid,),
        in_specs=[row_spec, row_spec, col_spec, col_spec, w_spec, w_spec],
        out_specs=row_spec,
        out_shape=out_sd,
    )(ab, ac, rb, rc, wb, wc)


def _scale2_body(e_ref, rb_ref, rc_ref, ob_ref, oc_ref):
    x = e_ref[...] * 0.01
    ob_ref[...] = x * rb_ref[...]
    oc_ref[...] = x * rc_ref[...]


def _scale2(emb, rb, rc):
    n, d = emb.shape
    grid = n // _ROW_BLK
    row_spec = pl.BlockSpec((_ROW_BLK, d), lambda i: (i, 0))
    col_spec = pl.BlockSpec((_ROW_BLK, 1), lambda i: (i, 0))
    out_sd = jax.ShapeDtypeStruct((n, d), jnp.float32)
    return pl.pallas_call(
        _scale2_body,
        grid=(grid,),
        in_specs=[row_spec, col_spec, col_spec],
        out_specs=[row_spec, row_spec],
        out_shape=[out_sd, out_sd],
    )(emb, rb, rc)


def kernel(buy_src, buy_dst, cart_src, cart_dst, pv_src, pv_dst, e_type,
           user_emb, item_emb, edges_emb, W1, W2, W3, W4, EW1, EW2, EW3, EW4):
    nu, d = user_emb.shape
    ni = item_emb.shape[0]
    ef = edges_emb[e_type] * 0.01
    vb, vc = ef[0], ef[2]

    pu = 4 * _N_CORES * _SLAB
    pi = 2 * _N_CORES * _SLAB
    assert pu >= nu and pi >= ni and pu % _ROW_BLK == 0 and pi % _ROW_BLK == 0

    ones = jnp.ones(buy_src.shape[0], jnp.float32)
    duB = jnp.maximum(jax.ops.segment_sum(ones, buy_src, num_segments=nu), 1.0)
    diB = jnp.maximum(jax.ops.segment_sum(ones, buy_dst, num_segments=ni), 1.0)
    duC = jnp.maximum(jax.ops.segment_sum(ones, cart_src, num_segments=nu), 1.0)
    diC = jnp.maximum(jax.ops.segment_sum(ones, cart_dst, num_segments=ni), 1.0)
    rbu = jnp.pad(lax.rsqrt(duB), (0, pu - nu), constant_values=1.0)[:, None]
    rcu = jnp.pad(lax.rsqrt(duC), (0, pu - nu), constant_values=1.0)[:, None]
    rbi = jnp.pad(lax.rsqrt(diB), (0, pi - ni), constant_values=1.0)[:, None]
    rci = jnp.pad(lax.rsqrt(diC), (0, pi - ni), constant_values=1.0)[:, None]

    eu = jnp.pad(user_emb, ((0, pu - nu), (0, 0)))
    ei = jnp.pad(item_emb, ((0, pi - ni), (0, 0)))
    sfB, sfC = _scale2(eu, rbu, rcu)
    dfB, dfC = _scale2(ei, rbi, rci)

    zeros_h = jnp.zeros((_STRIPE, d), jnp.float32)

    out_u = out_i = None
    for li, (W, EW) in enumerate(((W1, EW1), (W2, EW2), (W3, EW3), (W4, EW4))):
        wb_f = vb[:, None] * W
        wc_f = vc[:, None] * W
        aggU_b = _sc_agg(dfB, buy_dst, buy_src, zeros_h, 4)
        aggI_b = _sc_agg(sfB, buy_src, buy_dst, zeros_h, 2)
        aggU_c = _sc_agg(dfC, cart_dst, cart_src, zeros_h, 4)
        aggI_c = _sc_agg(sfC, cart_src, cart_dst, zeros_h, 2)
        if li < 3:
            sfB, sfC = _combine(aggU_b, aggU_c, rbu, rcu, wb_f, wc_f, True)
            dfB, dfC = _combine(aggI_b, aggI_c, rbi, rci, wb_f, wc_f, True)
            vb, vc = vb @ EW, vc @ EW
        else:
            out_u = _combine(aggU_b, aggU_c, rbu, rcu, wb_f, wc_f, False)
            out_i = _combine(aggI_b, aggI_c, rbi, rci, wb_f, wc_f, False)
    return out_u[:nu], out_i[:ni]
